# baseline (speedup 1.0000x reference)
import functools

import jax
import jax.numpy as jnp
from jax import lax
from jax.experimental import pallas as pl
from jax.experimental.pallas import tpu as pltpu
from jax.experimental.pallas import tpu_sc as plsc

VOCAB = 1000000
SEQ_LEN = 200
D_MODEL = 64
BATCH = 4096
NGB = 4
NOB = 1


def _sc_call(idxT, tab2, pos_enc):
    info = plsc.get_sparse_core_info()
    nc, ns = info.num_cores, info.num_subcores
    nw = nc * ns
    bcol = BATCH // nw
    ncc = bcol // 16

    mesh = plsc.VectorSubcoreMesh(core_axis_name="c", subcore_axis_name="s")

    scratch = (
        [pltpu.VMEM((SEQ_LEN, bcol), jnp.int32)]
        + [pltpu.VMEM((bcol,), jnp.int32) for _ in range(NGB)]
        + [pltpu.VMEM((bcol, 2 * D_MODEL), jnp.float32) for _ in range(NGB)]
        + [pltpu.VMEM((D_MODEL, bcol), jnp.float32) for _ in range(NOB)]
        + [pltpu.VMEM((SEQ_LEN, D_MODEL), jnp.float32)]
        + [pltpu.SemaphoreType.DMA for _ in range(NGB + NOB)]
    )

    @functools.partial(
        pl.kernel,
        out_type=jax.ShapeDtypeStruct((SEQ_LEN, D_MODEL, BATCH), jnp.float32),
        mesh=mesh,
        scratch_types=scratch,
        compiler_params=pltpu.CompilerParams(
            use_tc_tiling_on_sc=True, needs_layout_passes=False),
    )
    def k(idxT_hbm, tab2_hbm, pos_hbm, out_hbm, idx_v, *rest):
        grp = rest[:NGB]
        gath = rest[NGB:2 * NGB]
        outt = rest[2 * NGB:2 * NGB + NOB]
        pos_v = rest[2 * NGB + NOB]
        gsem = rest[2 * NGB + NOB + 1:2 * NGB + NOB + 1 + NGB]
        osem = rest[2 * NGB + NOB + 1 + NGB:]

        wid = lax.axis_index("s") * nc + lax.axis_index("c")
        b0 = pl.multiple_of(wid * bcol, bcol)

        pltpu.sync_copy(idxT_hbm.at[:, pl.ds(b0, bcol)], idx_v)
        pltpu.sync_copy(pos_hbm, pos_v)

        def issue_gather(s, p):
            for cc in range(ncc):
                v = idx_v[s, pl.ds(cc * 16, 16)]
                grp[p][pl.ds(cc * 16, 16)] = lax.shift_right_logical(v, 1)
            pltpu.async_copy(tab2_hbm.at[grp[p]], gath[p], gsem[p])

        def wait_gather(p):
            pltpu.make_async_copy(tab2_hbm.at[grp[p]], gath[p], gsem[p]).wait()

        def issue_out(s, q):
            pltpu.async_copy(outt[q], out_hbm.at[s, :, pl.ds(b0, bcol)],
                             osem[q])

        def wait_out(s, q):
            pltpu.make_async_copy(outt[q],
                                  out_hbm.at[s, :, pl.ds(b0, bcol)],
                                  osem[q]).wait()

        def compute(s, p, q):
            offs, kvecs = [], []
            for cc in range(ncc):
                v = idx_v[s, pl.ds(cc * 16, 16)]
                offs.append(lax.shift_left(lax.bitwise_and(v, 1), 6))
                kvecs.append(lax.iota(jnp.int32, 16) + cc * 16)
            sbc = lax.broadcast(s, (16,))

            def dbody(d, carry):
                offs_c, kvecs_c = carry
                pv = plsc.load_gather(pos_v, [sbc, lax.broadcast(d, (16,))])
                for cc in range(ncc):
                    ovec = offs_c[cc] + d
                    vals = plsc.load_gather(gath[p], [kvecs_c[cc], ovec])
                    outt[q][d, pl.ds(cc * 16, 16)] = vals + pv
                return carry
            lax.fori_loop(0, D_MODEL, dbody, (tuple(offs), tuple(kvecs)))

        def step(s, p, q, wait_o, issue_g):
            wait_gather(p)
            if wait_o:
                wait_out(s - NOB, q)
            compute(s, p, q)
            if issue_g:
                issue_gather(s + NGB, p)
            issue_out(s, q)

        for p in range(NGB):
            issue_gather(p, p)
        for s in range(NGB):
            step(s, s % NGB, s % NOB, wait_o=(s >= NOB), issue_g=True)

        def sbody(blk, carry):
            s0 = blk * NGB
            for j in range(NGB):
                step(s0 + j, j, j % NOB, wait_o=True, issue_g=True)
            return carry
        lax.fori_loop(1, SEQ_LEN // NGB - 1, sbody, 0)

        for j in range(NGB):
            s = SEQ_LEN - NGB + j
            step(s, s % NGB, s % NOB, wait_o=True, issue_g=False)
        for s in range(SEQ_LEN - NOB, SEQ_LEN):
            wait_out(s, s % NOB)

    return k(idxT, tab2, pos_enc)


def kernel(indices, table, pos_enc):
    idxT = indices.T.astype(jnp.int32)
    tab2 = table.reshape(VOCAB // 2, 2 * D_MODEL)
    out = _sc_call(idxT, tab2, pos_enc)
    return out.transpose(2, 0, 1)


# speedup vs baseline: 1.0399x; 1.0399x over previous
import functools

import jax
import jax.numpy as jnp
from jax import lax
from jax.experimental import pallas as pl
from jax.experimental.pallas import tpu as pltpu
from jax.experimental.pallas import tpu_sc as plsc

VOCAB = 1000000
SEQ_LEN = 200
D_MODEL = 64
BATCH = 4096
NGB = 2
NOB = 2
DUNROLL = 4


def _sc_call(idxT, tab2, pos_enc):
    info = plsc.get_sparse_core_info()
    nc, ns = info.num_cores, info.num_subcores
    nw = nc * ns
    bcol = BATCH // nw
    ncc = bcol // 16

    mesh = plsc.VectorSubcoreMesh(core_axis_name="c", subcore_axis_name="s")

    scratch = (
        [pltpu.VMEM((SEQ_LEN, bcol), jnp.int32)]
        + [pltpu.VMEM((bcol,), jnp.int32) for _ in range(NGB)]
        + [pltpu.VMEM((bcol, 2 * D_MODEL), jnp.float32) for _ in range(NGB)]
        + [pltpu.VMEM((D_MODEL, bcol), jnp.float32) for _ in range(NOB)]
        + [pltpu.VMEM((SEQ_LEN, D_MODEL), jnp.float32)]
        + [pltpu.SemaphoreType.DMA for _ in range(NGB + NOB)]
    )

    @functools.partial(
        pl.kernel,
        out_type=jax.ShapeDtypeStruct((SEQ_LEN, D_MODEL, BATCH), jnp.float32),
        mesh=mesh,
        scratch_types=scratch,
        compiler_params=pltpu.CompilerParams(
            use_tc_tiling_on_sc=True, needs_layout_passes=False),
    )
    def k(idxT_hbm, tab2_hbm, pos_hbm, out_hbm, idx_v, *rest):
        grp = rest[:NGB]
        gath = rest[NGB:2 * NGB]
        outt = rest[2 * NGB:2 * NGB + NOB]
        pos_v = rest[2 * NGB + NOB]
        gsem = rest[2 * NGB + NOB + 1:2 * NGB + NOB + 1 + NGB]
        osem = rest[2 * NGB + NOB + 1 + NGB:]

        wid = lax.axis_index("s") * nc + lax.axis_index("c")
        b0 = pl.multiple_of(wid * bcol, bcol)

        pltpu.sync_copy(idxT_hbm.at[:, pl.ds(b0, bcol)], idx_v)
        pltpu.sync_copy(pos_hbm, pos_v)

        def issue_gather(s, p):
            for cc in range(ncc):
                v = idx_v[s, pl.ds(cc * 16, 16)]
                grp[p][pl.ds(cc * 16, 16)] = lax.shift_right_logical(v, 1)
            pltpu.async_copy(tab2_hbm.at[grp[p]], gath[p], gsem[p])

        def wait_gather(p):
            pltpu.make_async_copy(tab2_hbm.at[grp[p]], gath[p], gsem[p]).wait()

        def issue_out(s, q):
            pltpu.async_copy(outt[q], out_hbm.at[s, :, pl.ds(b0, bcol)],
                             osem[q])

        def wait_out(s, q):
            pltpu.make_async_copy(outt[q],
                                  out_hbm.at[s, :, pl.ds(b0, bcol)],
                                  osem[q]).wait()

        def compute(s, p, q):
            offs, kvecs = [], []
            for cc in range(ncc):
                v = idx_v[s, pl.ds(cc * 16, 16)]
                offs.append(lax.shift_left(lax.bitwise_and(v, 1), 6))
                kvecs.append(lax.iota(jnp.int32, 16) + cc * 16)
            sbc = lax.broadcast(s, (16,))

            def dbody(t, carry):
                offs_c, kvecs_c = carry
                d0 = t * DUNROLL
                pvs = [
                    plsc.load_gather(
                        pos_v, [sbc, lax.broadcast(d0 + u, (16,))])
                    for u in range(DUNROLL)
                ]
                for cc in range(ncc):
                    for u in range(DUNROLL):
                        vals = plsc.load_gather(
                            gath[p], [kvecs_c[cc], offs_c[cc] + (d0 + u)])
                        outt[q][d0 + u, pl.ds(cc * 16, 16)] = vals + pvs[u]
                return carry
            lax.fori_loop(0, D_MODEL // DUNROLL, dbody,
                          (tuple(offs), tuple(kvecs)))

        def step(s, p, q, wait_o, issue_g):
            wait_gather(p)
            if wait_o:
                wait_out(s - NOB, q)
            compute(s, p, q)
            if issue_g:
                issue_gather(s + NGB, p)
            issue_out(s, q)

        for p in range(NGB):
            issue_gather(p, p)
        for s in range(NGB):
            step(s, s % NGB, s % NOB, wait_o=(s >= NOB), issue_g=True)

        def sbody(blk, carry):
            s0 = blk * NGB
            for j in range(NGB):
                step(s0 + j, j, j % NOB, wait_o=True, issue_g=True)
            return carry
        lax.fori_loop(1, SEQ_LEN // NGB - 1, sbody, 0)

        for j in range(NGB):
            s = SEQ_LEN - NGB + j
            step(s, s % NGB, s % NOB, wait_o=True, issue_g=False)
        for s in range(SEQ_LEN - NOB, SEQ_LEN):
            wait_out(s, s % NOB)

    return k(idxT, tab2, pos_enc)


def kernel(indices, table, pos_enc):
    idxT = indices.T.astype(jnp.int32)
    tab2 = table.reshape(VOCAB // 2, 2 * D_MODEL)
    out = _sc_call(idxT, tab2, pos_enc)
    return out.transpose(2, 0, 1)
